# baseline (speedup 1.0000x reference)
import functools

import jax
import jax.numpy as jnp
from jax import lax
from jax.experimental import pallas as pl
from jax.experimental.pallas import tpu as pltpu
from jax.experimental.pallas import tpu_sc as plsc

B = 16384
D = 128
H1 = 256
H2 = 64

NC = 2
NS = 16
NW = NC * NS

CH = 256


@functools.cache
def _make_gather(nb):
    bpw = nb // NW
    ch = min(CH, bpw)
    nt = (2 * bpw) // ch
    mesh = plsc.VectorSubcoreMesh(core_axis_name="c", subcore_axis_name="s")

    @functools.partial(
        pl.kernel,
        mesh=mesh,
        out_type=[
            jax.ShapeDtypeStruct((nb, D), jnp.float32),
            jax.ShapeDtypeStruct((nb, D), jnp.float32),
        ],
        scratch_types=[
            pltpu.VMEM((bpw,), jnp.int32),
            pltpu.VMEM((bpw,), jnp.int32),
            pltpu.VMEM((ch, D), jnp.float32),
            pltpu.VMEM((ch, D), jnp.float32),
            pltpu.SemaphoreType.DMA,
            pltpu.SemaphoreType.DMA,
            pltpu.SemaphoreType.DMA,
            pltpu.SemaphoreType.DMA,
        ],
    )
    def _gather_sc(uid_hbm, mid_hbm, ut_hbm, mt_hbm, eu_hbm, ev_hbm,
                   uidx_v, midx_v, bufa, bufb, sg0, sg1, sw0, sw1):
        wid = lax.axis_index("s") * NC + lax.axis_index("c")
        base = wid * bpw
        pltpu.sync_copy(uid_hbm.at[pl.ds(base, bpw)], uidx_v)
        pltpu.sync_copy(mid_hbm.at[pl.ds(base, bpw)], midx_v)
        bufs = (bufa, bufb)
        sgs = (sg0, sg1)
        sws = (sw0, sw1)
        tasks = []
        for idx_v, tab, out in ((uidx_v, ut_hbm, eu_hbm),
                                (midx_v, mt_hbm, ev_hbm)):
            for c in range(bpw // ch):
                tasks.append((idx_v, tab, out, c * ch))

        def issue_gather(k):
            idx_v, tab, _, off = tasks[k]
            b = k % 2
            return pltpu.async_copy(
                tab.at[idx_v.at[pl.ds(off, ch)]], bufs[b], sgs[b])

        gcp = [None] * nt
        wcp = [None] * nt
        gcp[0] = issue_gather(0)
        for k in range(nt):
            b = k % 2
            if k + 1 < nt:
                if k >= 1:
                    wcp[k - 1].wait()
                gcp[k + 1] = issue_gather(k + 1)
            gcp[k].wait()
            _, _, out, off = tasks[k]
            wcp[k] = pltpu.async_copy(
                bufs[b], out.at[pl.ds(base + off, ch)], sws[b])
        if nt >= 2:
            wcp[nt - 2].wait()
        wcp[nt - 1].wait()

    return _gather_sc


BLK = 2048


def _mlp_body(eu_ref, ev_ref, w1a_ref, w1b_ref, b1_ref, w2_ref, b2_ref,
              w3t_ref, b3_ref, o_ref):
    h = jnp.dot(eu_ref[...], w1a_ref[...], preferred_element_type=jnp.float32)
    h = h + jnp.dot(ev_ref[...], w1b_ref[...], preferred_element_type=jnp.float32)
    h = jnp.maximum(h + b1_ref[...], 0.0)
    h = jnp.dot(h, w2_ref[...], preferred_element_type=jnp.float32)
    h = jnp.maximum(h + b2_ref[...], 0.0)
    o_ref[...] = jnp.sum(h * w3t_ref[...], axis=1) + b3_ref[0, 0]


def _mlp(eu, ev, w1a, w1b, b1, w2, b2, w3t, b3):
    nb = eu.shape[0]
    blk = min(BLK, nb)
    grid = (nb // blk,)
    full = lambda i: (0, 0)
    return pl.pallas_call(
        _mlp_body,
        grid=grid,
        in_specs=[
            pl.BlockSpec((blk, D), lambda i: (i, 0)),
            pl.BlockSpec((blk, D), lambda i: (i, 0)),
            pl.BlockSpec((D, H1), full),
            pl.BlockSpec((D, H1), full),
            pl.BlockSpec((1, H1), full),
            pl.BlockSpec((H1, H2), full),
            pl.BlockSpec((1, H2), full),
            pl.BlockSpec((1, H2), full),
            pl.BlockSpec((1, 1), full),
        ],
        out_specs=pl.BlockSpec((blk,), lambda i: (i,)),
        out_shape=jax.ShapeDtypeStruct((nb,), jnp.float32),
    )(eu, ev, w1a, w1b, b1, w2, b2, w3t, b3)


NCHUNK = 2


def kernel(user_id, movie_id, user_table, movie_table, W1, b1, W2, b2, W3, b3):
    uid = user_id.astype(jnp.int32)
    mid = movie_id.astype(jnp.int32)
    w1a, w1b = W1[:D], W1[D:]
    w3t = W3.reshape(1, H2)
    b1r, b2r, b3r = b1.reshape(1, H1), b2.reshape(1, H2), b3.reshape(1, 1)
    nb = B // NCHUNK
    gather = _make_gather(nb)
    outs = []
    embs = [gather(uid[i * nb:(i + 1) * nb], mid[i * nb:(i + 1) * nb],
                   user_table, movie_table) for i in range(NCHUNK)]
    for eu, ev in embs:
        outs.append(_mlp(eu, ev, w1a, w1b, b1r, W2, b2r, w3t, b3r))
    return jnp.concatenate(outs, axis=0).reshape(B, 1)


# speedup vs baseline: 1.0919x; 1.0919x over previous
import functools

import jax
import jax.numpy as jnp
from jax import lax
from jax.experimental import pallas as pl
from jax.experimental.pallas import tpu as pltpu
from jax.experimental.pallas import tpu_sc as plsc

B = 16384
D = 128
H1 = 256
H2 = 64

NC = 2
NS = 16
NW = NC * NS

CH = 256


@functools.cache
def _make_gather(nb):
    bpw = nb // NW
    ch = min(CH, bpw)
    nt = (2 * bpw) // ch
    mesh = plsc.VectorSubcoreMesh(core_axis_name="c", subcore_axis_name="s")

    @functools.partial(
        pl.kernel,
        mesh=mesh,
        out_type=[
            jax.ShapeDtypeStruct((nb, D), jnp.float32),
            jax.ShapeDtypeStruct((nb, D), jnp.float32),
        ],
        scratch_types=[
            pltpu.VMEM((bpw,), jnp.int32),
            pltpu.VMEM((bpw,), jnp.int32),
            pltpu.VMEM((ch, D), jnp.float32),
            pltpu.VMEM((ch, D), jnp.float32),
            pltpu.SemaphoreType.DMA,
            pltpu.SemaphoreType.DMA,
            pltpu.SemaphoreType.DMA,
            pltpu.SemaphoreType.DMA,
        ],
    )
    def _gather_sc(uid_hbm, mid_hbm, ut_hbm, mt_hbm, eu_hbm, ev_hbm,
                   uidx_v, midx_v, bufa, bufb, sg0, sg1, sw0, sw1):
        wid = lax.axis_index("s") * NC + lax.axis_index("c")
        base = wid * bpw
        pltpu.sync_copy(uid_hbm.at[pl.ds(base, bpw)], uidx_v)
        pltpu.sync_copy(mid_hbm.at[pl.ds(base, bpw)], midx_v)
        bufs = (bufa, bufb)
        sgs = (sg0, sg1)
        sws = (sw0, sw1)
        tasks = []
        for idx_v, tab, out in ((uidx_v, ut_hbm, eu_hbm),
                                (midx_v, mt_hbm, ev_hbm)):
            for c in range(bpw // ch):
                tasks.append((idx_v, tab, out, c * ch))

        def issue_gather(k):
            idx_v, tab, _, off = tasks[k]
            b = k % 2
            return pltpu.async_copy(
                tab.at[idx_v.at[pl.ds(off, ch)]], bufs[b], sgs[b])

        gcp = [None] * nt
        wcp = [None] * nt
        gcp[0] = issue_gather(0)
        for k in range(nt):
            b = k % 2
            if k + 1 < nt:
                if k >= 1:
                    wcp[k - 1].wait()
                gcp[k + 1] = issue_gather(k + 1)
            gcp[k].wait()
            _, _, out, off = tasks[k]
            wcp[k] = pltpu.async_copy(
                bufs[b], out.at[pl.ds(base + off, ch)], sws[b])
        if nt >= 2:
            wcp[nt - 2].wait()
        wcp[nt - 1].wait()

    return _gather_sc


BLK = 1024


def _mlp_body(eu_ref, ev_ref, w1a_ref, w1b_ref, b1_ref, w2_ref, b2_ref,
              w3t_ref, b3_ref, o_ref):
    h = jnp.dot(eu_ref[...], w1a_ref[...], preferred_element_type=jnp.float32)
    h = h + jnp.dot(ev_ref[...], w1b_ref[...], preferred_element_type=jnp.float32)
    h = jnp.maximum(h + b1_ref[...], 0.0)
    h = jnp.dot(h, w2_ref[...], preferred_element_type=jnp.float32)
    h = jnp.maximum(h + b2_ref[...], 0.0)
    o_ref[...] = jnp.dot(h, w3t_ref[...], preferred_element_type=jnp.float32) + b3_ref[...]


def _mlp(eu, ev, w1a, w1b, b1, w2, b2, w3t, b3):
    nb = eu.shape[0]
    blk = min(BLK, nb)
    grid = (nb // blk,)
    full = lambda i: (0, 0)
    return pl.pallas_call(
        _mlp_body,
        grid=grid,
        in_specs=[
            pl.BlockSpec((blk, D), lambda i: (i, 0)),
            pl.BlockSpec((blk, D), lambda i: (i, 0)),
            pl.BlockSpec((D, H1), full),
            pl.BlockSpec((D, H1), full),
            pl.BlockSpec((1, H1), full),
            pl.BlockSpec((H1, H2), full),
            pl.BlockSpec((1, H2), full),
            pl.BlockSpec((H2, 1), full),
            pl.BlockSpec((1, 1), full),
        ],
        out_specs=pl.BlockSpec((blk, 1), lambda i: (i, 0)),
        out_shape=jax.ShapeDtypeStruct((nb, 1), jnp.float32),
    )(eu, ev, w1a, w1b, b1, w2, b2, w3t, b3)


NCHUNK = 2


def kernel(user_id, movie_id, user_table, movie_table, W1, b1, W2, b2, W3, b3):
    uid = user_id.astype(jnp.int32)
    mid = movie_id.astype(jnp.int32)
    w1a, w1b = W1[:D], W1[D:]
    w3t = W3
    b1r, b2r, b3r = b1.reshape(1, H1), b2.reshape(1, H2), b3.reshape(1, 1)
    nb = B // NCHUNK
    gather = _make_gather(nb)
    outs = []
    embs = [gather(uid[i * nb:(i + 1) * nb], mid[i * nb:(i + 1) * nb],
                   user_table, movie_table) for i in range(NCHUNK)]
    for eu, ev in embs:
        outs.append(_mlp(eu, ev, w1a, w1b, b1r, W2, b2r, w3t, b3r))
    return jnp.concatenate(outs, axis=0)


# speedup vs baseline: 1.1721x; 1.0735x over previous
import functools

import jax
import jax.numpy as jnp
from jax import lax
from jax.experimental import pallas as pl
from jax.experimental.pallas import tpu as pltpu
from jax.experimental.pallas import tpu_sc as plsc

B = 16384
D = 128
H1 = 256
H2 = 64

NC = 2
NS = 16
NW = NC * NS

CH = 256


@functools.cache
def _make_gather(nb):
    bpw = nb // NW
    ch = min(CH, bpw)
    nt = (2 * bpw) // ch
    mesh = plsc.VectorSubcoreMesh(core_axis_name="c", subcore_axis_name="s")

    @functools.partial(
        pl.kernel,
        mesh=mesh,
        out_type=[
            jax.ShapeDtypeStruct((nb, D), jnp.float32),
            jax.ShapeDtypeStruct((nb, D), jnp.float32),
        ],
        scratch_types=[
            pltpu.VMEM((bpw,), jnp.int32),
            pltpu.VMEM((bpw,), jnp.int32),
            pltpu.VMEM((ch, D), jnp.float32),
            pltpu.VMEM((ch, D), jnp.float32),
            pltpu.SemaphoreType.DMA,
            pltpu.SemaphoreType.DMA,
            pltpu.SemaphoreType.DMA,
            pltpu.SemaphoreType.DMA,
        ],
    )
    def _gather_sc(uid_hbm, mid_hbm, ut_hbm, mt_hbm, eu_hbm, ev_hbm,
                   uidx_v, midx_v, bufa, bufb, sg0, sg1, sw0, sw1):
        wid = lax.axis_index("s") * NC + lax.axis_index("c")
        base = wid * bpw
        pltpu.sync_copy(uid_hbm.at[pl.ds(base, bpw)], uidx_v)
        pltpu.sync_copy(mid_hbm.at[pl.ds(base, bpw)], midx_v)
        bufs = (bufa, bufb)
        sgs = (sg0, sg1)
        sws = (sw0, sw1)
        tasks = []
        for idx_v, tab, out in ((uidx_v, ut_hbm, eu_hbm),
                                (midx_v, mt_hbm, ev_hbm)):
            for c in range(bpw // ch):
                tasks.append((idx_v, tab, out, c * ch))

        def issue_gather(k):
            idx_v, tab, _, off = tasks[k]
            b = k % 2
            return pltpu.async_copy(
                tab.at[idx_v.at[pl.ds(off, ch)]], bufs[b], sgs[b])

        gcp = [None] * nt
        wcp = [None] * nt
        gcp[0] = issue_gather(0)
        for k in range(nt):
            b = k % 2
            if k + 1 < nt:
                if k >= 1:
                    wcp[k - 1].wait()
                gcp[k + 1] = issue_gather(k + 1)
            gcp[k].wait()
            _, _, out, off = tasks[k]
            wcp[k] = pltpu.async_copy(
                bufs[b], out.at[pl.ds(base + off, ch)], sws[b])
        if nt >= 2:
            wcp[nt - 2].wait()
        wcp[nt - 1].wait()

    return _gather_sc


BLK = 4096


def _mlp_body(eu_ref, ev_ref, w1a_ref, w1b_ref, b1_ref, w2_ref, b2_ref,
              w3t_ref, b3_ref, o_ref):
    h = jnp.dot(eu_ref[...], w1a_ref[...], preferred_element_type=jnp.float32)
    h = h + jnp.dot(ev_ref[...], w1b_ref[...], preferred_element_type=jnp.float32)
    h = jnp.maximum(h + b1_ref[...], 0.0)
    h = jnp.dot(h, w2_ref[...], preferred_element_type=jnp.float32)
    h = jnp.maximum(h + b2_ref[...], 0.0)
    o_ref[...] = jnp.dot(h, w3t_ref[...], preferred_element_type=jnp.float32) + b3_ref[...]


def _mlp(eu, ev, w1a, w1b, b1, w2, b2, w3t, b3):
    nb = eu.shape[0]
    blk = min(BLK, nb)
    grid = (nb // blk,)
    full = lambda i: (0, 0)
    return pl.pallas_call(
        _mlp_body,
        grid=grid,
        in_specs=[
            pl.BlockSpec((blk, D), lambda i: (i, 0)),
            pl.BlockSpec((blk, D), lambda i: (i, 0)),
            pl.BlockSpec((D, H1), full),
            pl.BlockSpec((D, H1), full),
            pl.BlockSpec((1, H1), full),
            pl.BlockSpec((H1, H2), full),
            pl.BlockSpec((1, H2), full),
            pl.BlockSpec((H2, 1), full),
            pl.BlockSpec((1, 1), full),
        ],
        out_specs=pl.BlockSpec((blk, 1), lambda i: (i, 0)),
        out_shape=jax.ShapeDtypeStruct((nb, 1), jnp.float32),
    )(eu, ev, w1a, w1b, b1, w2, b2, w3t, b3)


NCHUNK = 2


def kernel(user_id, movie_id, user_table, movie_table, W1, b1, W2, b2, W3, b3):
    uid = user_id.astype(jnp.int32)
    mid = movie_id.astype(jnp.int32)
    w1a, w1b = W1[:D], W1[D:]
    w3t = W3
    b1r, b2r, b3r = b1.reshape(1, H1), b2.reshape(1, H2), b3.reshape(1, 1)
    nb = B // NCHUNK
    gather = _make_gather(nb)
    outs = []
    embs = [gather(uid[i * nb:(i + 1) * nb], mid[i * nb:(i + 1) * nb],
                   user_table, movie_table) for i in range(NCHUNK)]
    for eu, ev in embs:
        outs.append(_mlp(eu, ev, w1a, w1b, b1r, W2, b2r, w3t, b3r))
    return jnp.concatenate(outs, axis=0)


# speedup vs baseline: 1.1786x; 1.0055x over previous
import functools

import jax
import jax.numpy as jnp
from jax import lax
from jax.experimental import pallas as pl
from jax.experimental.pallas import tpu as pltpu
from jax.experimental.pallas import tpu_sc as plsc

B = 16384
D = 128
H1 = 256
H2 = 64

NC = 2
NS = 16
NW = NC * NS

CH = 256


@functools.cache
def _make_gather(nb):
    bpw = nb // NW
    ch = min(CH, bpw)
    nt = (2 * bpw) // ch
    mesh = plsc.VectorSubcoreMesh(core_axis_name="c", subcore_axis_name="s")

    @functools.partial(
        pl.kernel,
        mesh=mesh,
        out_type=[
            jax.ShapeDtypeStruct((nb, D), jnp.float32),
            jax.ShapeDtypeStruct((nb, D), jnp.float32),
        ],
        scratch_types=[
            pltpu.VMEM((bpw,), jnp.int32),
            pltpu.VMEM((bpw,), jnp.int32),
            pltpu.VMEM((ch, D), jnp.float32),
            pltpu.VMEM((ch, D), jnp.float32),
            pltpu.SemaphoreType.DMA,
            pltpu.SemaphoreType.DMA,
            pltpu.SemaphoreType.DMA,
            pltpu.SemaphoreType.DMA,
        ],
    )
    def _gather_sc(uid_hbm, mid_hbm, ut_hbm, mt_hbm, eu_hbm, ev_hbm,
                   uidx_v, midx_v, bufa, bufb, sg0, sg1, sw0, sw1):
        wid = lax.axis_index("s") * NC + lax.axis_index("c")
        base = wid * bpw
        pltpu.sync_copy(uid_hbm.at[pl.ds(base, bpw)], uidx_v)
        pltpu.sync_copy(mid_hbm.at[pl.ds(base, bpw)], midx_v)
        bufs = (bufa, bufb)
        sgs = (sg0, sg1)
        sws = (sw0, sw1)
        tasks = []
        for idx_v, tab, out in ((uidx_v, ut_hbm, eu_hbm),
                                (midx_v, mt_hbm, ev_hbm)):
            for c in range(bpw // ch):
                tasks.append((idx_v, tab, out, c * ch))

        def issue_gather(k):
            idx_v, tab, _, off = tasks[k]
            b = k % 2
            return pltpu.async_copy(
                tab.at[idx_v.at[pl.ds(off, ch)]], bufs[b], sgs[b])

        gcp = [None] * nt
        wcp = [None] * nt
        gcp[0] = issue_gather(0)
        for k in range(nt):
            b = k % 2
            if k + 1 < nt:
                if k >= 1:
                    wcp[k - 1].wait()
                gcp[k + 1] = issue_gather(k + 1)
            gcp[k].wait()
            _, _, out, off = tasks[k]
            wcp[k] = pltpu.async_copy(
                bufs[b], out.at[pl.ds(base + off, ch)], sws[b])
        if nt >= 2:
            wcp[nt - 2].wait()
        wcp[nt - 1].wait()

    return _gather_sc


BLK = 4096


def _mlp_body(eu_ref, ev_ref, w1a_ref, w1b_ref, b1_ref, w2_ref, b2_ref,
              w3t_ref, b3_ref, o_ref):
    bf = jnp.bfloat16
    h = jnp.dot(eu_ref[...].astype(bf), w1a_ref[...].astype(bf),
                preferred_element_type=jnp.float32)
    h = h + jnp.dot(ev_ref[...].astype(bf), w1b_ref[...].astype(bf),
                    preferred_element_type=jnp.float32)
    h = jnp.maximum(h + b1_ref[...], 0.0)
    h = jnp.dot(h.astype(bf), w2_ref[...].astype(bf),
                preferred_element_type=jnp.float32)
    h = jnp.maximum(h + b2_ref[...], 0.0)
    o_ref[...] = jnp.dot(h, w3t_ref[...], preferred_element_type=jnp.float32) + b3_ref[...]


def _mlp(eu, ev, w1a, w1b, b1, w2, b2, w3t, b3):
    nb = eu.shape[0]
    blk = min(BLK, nb)
    grid = (nb // blk,)
    full = lambda i: (0, 0)
    return pl.pallas_call(
        _mlp_body,
        grid=grid,
        in_specs=[
            pl.BlockSpec((blk, D), lambda i: (i, 0)),
            pl.BlockSpec((blk, D), lambda i: (i, 0)),
            pl.BlockSpec((D, H1), full),
            pl.BlockSpec((D, H1), full),
            pl.BlockSpec((1, H1), full),
            pl.BlockSpec((H1, H2), full),
            pl.BlockSpec((1, H2), full),
            pl.BlockSpec((H2, 1), full),
            pl.BlockSpec((1, 1), full),
        ],
        out_specs=pl.BlockSpec((blk, 1), lambda i: (i, 0)),
        out_shape=jax.ShapeDtypeStruct((nb, 1), jnp.float32),
    )(eu, ev, w1a, w1b, b1, w2, b2, w3t, b3)


NCHUNK = 2


def kernel(user_id, movie_id, user_table, movie_table, W1, b1, W2, b2, W3, b3):
    uid = user_id.astype(jnp.int32)
    mid = movie_id.astype(jnp.int32)
    w1a, w1b = W1[:D], W1[D:]
    w3t = W3
    b1r, b2r, b3r = b1.reshape(1, H1), b2.reshape(1, H2), b3.reshape(1, 1)
    nb = B // NCHUNK
    gather = _make_gather(nb)
    outs = []
    embs = [gather(uid[i * nb:(i + 1) * nb], mid[i * nb:(i + 1) * nb],
                   user_table, movie_table) for i in range(NCHUNK)]
    for eu, ev in embs:
        outs.append(_mlp(eu, ev, w1a, w1b, b1r, W2, b2r, w3t, b3r))
    return jnp.concatenate(outs, axis=0)
